# baseline (speedup 1.0000x reference)
import functools

import jax
import jax.numpy as jnp
from jax import lax
from jax.experimental import pallas as pl
from jax.experimental.pallas import tpu as pltpu
from jax.experimental.pallas import tpu_sc as plsc

VOCAB = 100000
SEQ = 2048
BATCH = 4
EMBED = 1024

NC = 2
NS = 16
L = 16
NW = NC * NS

ROWS = BATCH * SEQ
RPW = ROWS // NW
CHUNK = 16
NCHUNK = RPW // CHUNK
NBUF = 4
NPOS = 2
NSLICE = EMBED // L
STATS = 32

_GATHER_DNUMS = lax.GatherDimensionNumbers(
    offset_dims=(), collapsed_slice_dims=(0,), start_index_map=(0,)
)


def _lane_sum(v):
    for sh in (8, 4, 2, 1):
        idx = lax.iota(jnp.int32, L) ^ sh
        v = v + lax.gather(
            v, idx[:, None], _GATHER_DNUMS, (1,),
            mode=lax.GatherScatterMode.PROMISE_IN_BOUNDS,
        )
    return v


_mesh = plsc.VectorSubcoreMesh(
    core_axis_name="c", subcore_axis_name="s", num_cores=NC, num_subcores=NS
)


@functools.partial(
    pl.kernel,
    out_type=(
        jax.ShapeDtypeStruct((ROWS, EMBED), jnp.float32),
        jax.ShapeDtypeStruct((ROWS, STATS), jnp.float32),
    ),
    mesh=_mesh,
    compiler_params=pltpu.CompilerParams(needs_layout_passes=False),
    scratch_types=[
        pltpu.VMEM((RPW,), jnp.int32),
        pltpu.VMEM((NBUF, CHUNK, EMBED), jnp.float32),
        pltpu.VMEM((NPOS, CHUNK, EMBED), jnp.float32),
        pltpu.VMEM((NBUF, CHUNK, STATS), jnp.float32),
        [pltpu.SemaphoreType.DMA] * NBUF,
        [pltpu.SemaphoreType.DMA] * NPOS,
        [pltpu.SemaphoreType.DMA] * NBUF,
        [pltpu.SemaphoreType.DMA] * NBUF,
    ],
)
def _sc_gather_stats(ids_hbm, table_hbm, pos_hbm, emb_hbm, stats_hbm,
                     idx_v, toks, poss, sbufs, gsems, psems, osems, ssems):
    wid = lax.axis_index("s") * NC + lax.axis_index("c")
    base = wid * RPW
    pos_base = lax.rem(base, SEQ)

    pltpu.sync_copy(ids_hbm.at[pl.ds(base, RPW)], idx_v)

    def start_tok(k, b):
        pltpu.async_copy(
            table_hbm.at[idx_v.at[pl.ds(k * CHUNK, CHUNK)]], toks.at[b],
            gsems[b],
        )

    def wait_tok(k, b):
        pltpu.make_async_copy(
            table_hbm.at[idx_v.at[pl.ds(k * CHUNK, CHUNK)]], toks.at[b],
            gsems[b],
        ).wait()

    def start_pos(k, pb):
        pltpu.async_copy(
            pos_hbm.at[pl.ds(pos_base + k * CHUNK, CHUNK)], poss.at[pb],
            psems[pb],
        )

    def wait_pos(k, pb):
        pltpu.make_async_copy(
            pos_hbm.at[pl.ds(pos_base + k * CHUNK, CHUNK)], poss.at[pb],
            psems[pb],
        ).wait()

    def compute(tok_buf, pos_buf, sbuf):
        def row_body(r, _):
            zero = jnp.zeros((L,), jnp.float32)
            s = [zero] * 4
            q = [zero] * 4
            for j in range(NSLICE):
                sl = pl.ds(j * L, L)
                v = tok_buf[r, sl] + pos_buf[r, sl]
                tok_buf[r, sl] = v
                s[j % 4] = s[j % 4] + v
                q[j % 4] = q[j % 4] + v * v
            s_tot = (s[0] + s[1]) + (s[2] + s[3])
            q_tot = (q[0] + q[1]) + (q[2] + q[3])
            mean_v = _lane_sum(s_tot) * (1.0 / EMBED)
            var_v = _lane_sum(q_tot) * (1.0 / EMBED) - mean_v * mean_v + 1e-5
            y = plsc.bitcast(
                jnp.int32(0x5F3759DF) - (plsc.bitcast(var_v, jnp.int32) >> 1),
                jnp.float32,
            )
            for _ in range(3):
                y = y * (1.5 - 0.5 * var_v * y * y)
            sbuf[r, pl.ds(0, L)] = mean_v
            sbuf[r, pl.ds(L, L)] = y
            return 0

        lax.fori_loop(0, CHUNK, row_body, 0)

    for b in range(3):
        start_tok(b, b)
    start_pos(0, 0)

    def step_body(step, _):
        for ph in range(4):
            m = step * 4 + ph
            b = ph
            pb = ph % NPOS
            wait_tok(m, b)
            wait_pos(m, pb)

            @pl.when(m + 1 < NCHUNK)
            def _():
                start_pos(m + 1, (ph + 1) % NPOS)

            compute(toks.at[b], poss.at[pb], sbufs.at[b])
            pltpu.async_copy(
                toks.at[b], emb_hbm.at[pl.ds(base + m * CHUNK, CHUNK)],
                osems[b],
            )
            pltpu.async_copy(
                sbufs.at[b], stats_hbm.at[pl.ds(base + m * CHUNK, CHUNK)],
                ssems[b],
            )
            b3 = (ph + 3) % NBUF

            @pl.when((m >= 1) & (m + 3 < NCHUNK))
            def _():
                k_prev = m - 1
                pltpu.make_async_copy(
                    toks.at[b3],
                    emb_hbm.at[pl.ds(base + k_prev * CHUNK, CHUNK)],
                    osems[b3],
                ).wait()
                pltpu.make_async_copy(
                    sbufs.at[b3],
                    stats_hbm.at[pl.ds(base + k_prev * CHUNK, CHUNK)],
                    ssems[b3],
                ).wait()

            @pl.when(m + 3 < NCHUNK)
            def _():
                start_tok(m + 3, b3)

        return 0

    lax.fori_loop(0, NCHUNK // 4, step_body, 0)

    for k in range(NCHUNK - 4, NCHUNK):
        b = k % NBUF
        pltpu.make_async_copy(
            toks.at[b], emb_hbm.at[pl.ds(base + k * CHUNK, CHUNK)], osems[b]
        ).wait()
        pltpu.make_async_copy(
            sbufs.at[b], stats_hbm.at[pl.ds(base + k * CHUNK, CHUNK)], ssems[b]
        ).wait()


BM = 2048


def _tc_ln_body(emb_ref, stats_ref, g_ref, b_ref, out_ref):
    e = emb_ref[...]
    mean = stats_ref[:, 0:1]
    rstd = stats_ref[:, L:L + 1]
    out_ref[...] = (e - mean) * rstd * g_ref[...] + b_ref[...]


_tc_ln = pl.pallas_call(
    _tc_ln_body,
    grid=(ROWS // BM,),
    in_specs=[
        pl.BlockSpec((BM, EMBED), lambda i: (i, 0)),
        pl.BlockSpec((BM, STATS), lambda i: (i, 0)),
        pl.BlockSpec((1, EMBED), lambda i: (0, 0)),
        pl.BlockSpec((1, EMBED), lambda i: (0, 0)),
    ],
    out_specs=pl.BlockSpec((BM, EMBED), lambda i: (i, 0)),
    out_shape=jax.ShapeDtypeStruct((ROWS, EMBED), jnp.float32),
)


def kernel(input_ids, token_table, pos_table, gamma, beta):
    flat_ids = input_ids.reshape(-1).astype(jnp.int32)
    emb, stats = _sc_gather_stats(flat_ids, token_table, pos_table)
    out = _tc_ln(emb, stats, gamma.reshape(1, EMBED), beta.reshape(1, EMBED))
    return out.reshape(BATCH, SEQ, EMBED)


# speedup vs baseline: 1.3075x; 1.3075x over previous
import functools

import jax
import jax.numpy as jnp
from jax import lax
from jax.experimental import pallas as pl
from jax.experimental.pallas import tpu as pltpu
from jax.experimental.pallas import tpu_sc as plsc

VOCAB = 100000
SEQ = 2048
BATCH = 4
EMBED = 1024

NC = 2
NS = 16
NW = NC * NS

ROWS = BATCH * SEQ
RPW = ROWS // NW
CHUNK = 32
NCHUNK = RPW // CHUNK
NBUF = 3

_mesh = plsc.VectorSubcoreMesh(
    core_axis_name="c", subcore_axis_name="s", num_cores=NC, num_subcores=NS
)


@functools.partial(
    pl.kernel,
    out_type=jax.ShapeDtypeStruct((ROWS, EMBED), jnp.float32),
    mesh=_mesh,
    compiler_params=pltpu.CompilerParams(needs_layout_passes=False),
    scratch_types=[
        pltpu.VMEM((RPW,), jnp.int32),
        pltpu.VMEM((NBUF, CHUNK, EMBED), jnp.float32),
        [pltpu.SemaphoreType.DMA] * NBUF,
        [pltpu.SemaphoreType.DMA] * NBUF,
    ],
)
def _sc_gather(ids_hbm, table_hbm, out_hbm, idx_v, bufs, gsems, osems):
    wid = lax.axis_index("s") * NC + lax.axis_index("c")
    base = wid * RPW
    pltpu.sync_copy(ids_hbm.at[pl.ds(base, RPW)], idx_v)

    def start(k, b):
        pltpu.async_copy(
            table_hbm.at[idx_v.at[pl.ds(k * CHUNK, CHUNK)]], bufs.at[b],
            gsems[b],
        )

    def wait(k, b):
        pltpu.make_async_copy(
            table_hbm.at[idx_v.at[pl.ds(k * CHUNK, CHUNK)]], bufs.at[b],
            gsems[b],
        ).wait()

    def start_out(k, b):
        pltpu.async_copy(
            bufs.at[b], out_hbm.at[pl.ds(base + k * CHUNK, CHUNK)], osems[b]
        )

    def wait_out(k, b):
        pltpu.make_async_copy(
            bufs.at[b], out_hbm.at[pl.ds(base + k * CHUNK, CHUNK)], osems[b]
        ).wait()

    for b in range(NBUF):
        start(b, b)
    for k in range(NCHUNK):
        b = k % NBUF
        wait(k, b)
        start_out(k, b)
        if k + NBUF < NCHUNK:
            wait_out(k, b)
            start(k + NBUF, b)
    for k in range(NCHUNK - NBUF, NCHUNK):
        wait_out(k, k % NBUF)


BM = 2048


def _tc_ln_body(emb_ref, pos_ref, g_ref, b_ref, out_ref):
    e = emb_ref[...] + pos_ref[...]
    mean = jnp.mean(e, axis=-1, keepdims=True)
    var = jnp.mean(e * e, axis=-1, keepdims=True) - mean * mean
    out_ref[...] = (e - mean) * lax.rsqrt(var + 1e-5) * g_ref[...] + b_ref[...]


_tc_ln = pl.pallas_call(
    _tc_ln_body,
    grid=(SEQ // BM, BATCH),
    in_specs=[
        pl.BlockSpec((BM, EMBED), lambda i, j: (j * (SEQ // BM) + i, 0)),
        pl.BlockSpec((BM, EMBED), lambda i, j: (i, 0)),
        pl.BlockSpec((1, EMBED), lambda i, j: (0, 0)),
        pl.BlockSpec((1, EMBED), lambda i, j: (0, 0)),
    ],
    out_specs=pl.BlockSpec((BM, EMBED), lambda i, j: (j * (SEQ // BM) + i, 0)),
    out_shape=jax.ShapeDtypeStruct((ROWS, EMBED), jnp.float32),
)


def kernel(input_ids, token_table, pos_table, gamma, beta):
    flat_ids = input_ids.reshape(-1).astype(jnp.int32)
    emb = _sc_gather(flat_ids, token_table)
    out = _tc_ln(emb, pos_table, gamma.reshape(1, EMBED), beta.reshape(1, EMBED))
    return out.reshape(BATCH, SEQ, EMBED)
